# baseline (speedup 1.0000x reference)
import jax
import jax.numpy as jnp
import numpy as np
from jax import lax
from jax.experimental import pallas as pl
from jax.experimental.pallas import tpu as pltpu

_KS0 = np.uint32(0)
_KS1 = np.uint32(42)
_KS2 = np.uint32(42 ^ 0x1BD11BDA)
_ROT_A = (13, 15, 26, 6)
_ROT_B = (17, 29, 16, 24)
_UNIF_LO = np.float32(1e-9)
_UNIF_SCALE = np.float32(np.float32(1.0) - np.float32(1e-9))


def _rotl(x, d):
    return (x << np.uint32(d)) | (x >> np.uint32(32 - d))


def _gumbel_from_counter(f):
    x0 = jnp.zeros_like(f) + _KS0
    x1 = f + _KS1
    kseq = ((_KS1, _KS2), (_KS2, _KS0), (_KS0, _KS1), (_KS1, _KS2), (_KS2, _KS0))
    rots = (_ROT_A, _ROT_B, _ROT_A, _ROT_B, _ROT_A)
    for i in range(5):
        for r in rots[i]:
            x0 = x0 + x1
            x1 = _rotl(x1, r)
            x1 = x0 ^ x1
        a, b = kseq[i]
        x0 = x0 + a
        x1 = x1 + b + np.uint32(i + 1)
    bits = x0 ^ x1
    fl = lax.bitcast_convert_type(
        (bits >> np.uint32(9)) | np.uint32(0x3F800000), jnp.float32
    ) - np.float32(1.0)
    u = jnp.maximum(_UNIF_LO, fl * _UNIF_SCALE + _UNIF_LO)
    return -jnp.log(-jnp.log(u))


def _argmax_body(lg_ref, idx_ref, ids_ref):
    i = pl.program_id(0)
    _, R, V = lg_ref.shape
    base = (i * np.int32(R * V)).astype(jnp.uint32)
    f = (
        lax.broadcasted_iota(jnp.uint32, (R, V), 0) * np.uint32(V)
        + lax.broadcasted_iota(jnp.uint32, (R, V), 1)
        + base
    )
    y = lg_ref[0] + _gumbel_from_counter(f)
    m = jnp.max(y, axis=1, keepdims=True)
    cols = lax.broadcasted_iota(jnp.int32, (R, V), 1)
    idx = jnp.min(jnp.where(y == m, cols, np.int32(2**30)), axis=1, keepdims=True)
    idx_ref[0] = jnp.broadcast_to(idx, idx_ref.shape[1:])
    ids_ref[0] = jnp.broadcast_to(idx, ids_ref.shape[1:]).astype(jnp.float32)


def _onehot_body(idx_ref, oh_ref):
    _, R, V = oh_ref.shape
    idxv = idx_ref[0, :, 0:1]
    cols = lax.broadcasted_iota(jnp.int32, (R, V), 1)
    oh_ref[0] = jnp.where(cols == idxv, np.float32(1.0), np.float32(0.0))


def kernel(logits):
    B, S, V = logits.shape
    R = 8
    idx, ids = pl.pallas_call(
        _argmax_body,
        grid=(S // R,),
        in_specs=[pl.BlockSpec((1, R, V), lambda i: (0, i, 0))],
        out_specs=[
            pl.BlockSpec((1, R, 128), lambda i: (0, i, 0)),
            pl.BlockSpec((1, R, 128), lambda i: (0, i, 0)),
        ],
        out_shape=[
            jax.ShapeDtypeStruct((B, S, 128), jnp.int32),
            jax.ShapeDtypeStruct((B, S, 128), jnp.float32),
        ],
        compiler_params=pltpu.CompilerParams(
            dimension_semantics=("parallel",)
        ),
    )(logits)
    R2 = 16
    onehot = pl.pallas_call(
        _onehot_body,
        grid=(S // R2,),
        in_specs=[pl.BlockSpec((1, R2, 128), lambda i: (0, i, 0))],
        out_specs=pl.BlockSpec((1, R2, V), lambda i: (0, i, 0)),
        out_shape=jax.ShapeDtypeStruct((B, S, V), jnp.float32),
        compiler_params=pltpu.CompilerParams(
            dimension_semantics=("parallel",)
        ),
    )(idx)
    return ids[:, :, 0], onehot


# speedup vs baseline: 1.9444x; 1.9444x over previous
import jax
import jax.numpy as jnp
import numpy as np
from jax import lax
from jax.experimental import pallas as pl
from jax.experimental.pallas import tpu as pltpu

_KS0 = np.uint32(0)
_KS1 = np.uint32(42)
_KS2 = np.uint32(42 ^ 0x1BD11BDA)
_ROT_A = (13, 15, 26, 6)
_ROT_B = (17, 29, 16, 24)
_UNIF_LO = np.float32(1e-9)
_UNIF_SCALE = np.float32(np.float32(1.0) - np.float32(1e-9))


def _rotl(x, d):
    return (x << np.uint32(d)) | (x >> np.uint32(32 - d))


def _gumbel_from_counter(f):
    x0 = jnp.zeros_like(f) + _KS0
    x1 = f + _KS1
    kseq = ((_KS1, _KS2), (_KS2, _KS0), (_KS0, _KS1), (_KS1, _KS2), (_KS2, _KS0))
    rots = (_ROT_A, _ROT_B, _ROT_A, _ROT_B, _ROT_A)
    for i in range(5):
        for r in rots[i]:
            x0 = x0 + x1
            x1 = _rotl(x1, r)
            x1 = x0 ^ x1
        a, b = kseq[i]
        x0 = x0 + a
        x1 = x1 + b + np.uint32(i + 1)
    bits = x0 ^ x1
    fl = lax.bitcast_convert_type(
        (bits >> np.uint32(9)) | np.uint32(0x3F800000), jnp.float32
    ) - np.float32(1.0)
    u = fl + _UNIF_LO
    return -jnp.log(-jnp.log(u))


_CW = 1024


def _argmax_body(lg_ref, idx_ref, ids_ref):
    i = pl.program_id(0)
    R, V = lg_ref.shape
    nc = (V + _CW - 1) // _CW
    base = (i * np.int32(R * V)).astype(jnp.uint32)
    fbase = (
        lax.broadcasted_iota(jnp.uint32, (R, _CW), 0) * np.uint32(V)
        + lax.broadcasted_iota(jnp.uint32, (R, _CW), 1)
        + base
    )
    ciota_i = lax.broadcasted_iota(jnp.int32, (R, _CW), 1)

    vmax = jnp.full((R, _CW), -jnp.inf, jnp.float32)
    vidx = jnp.zeros((R, _CW), jnp.int32)
    for c in range(nc):
        cb = min(c * _CW, V - _CW)
        y = lg_ref[:, cb : cb + _CW] + _gumbel_from_counter(fbase + np.uint32(cb))
        upd = y > vmax
        vmax = jnp.where(upd, y, vmax)
        vidx = jnp.where(upd, ciota_i + np.int32(cb), vidx)
    m = jnp.max(vmax, axis=1, keepdims=True)
    idx = jnp.min(
        jnp.where(vmax == m, vidx, np.int32(2**30)), axis=1, keepdims=True
    )
    idx_ref[:, :] = jnp.broadcast_to(idx, idx_ref.shape)
    ids_ref[:, :] = jnp.broadcast_to(idx, ids_ref.shape).astype(jnp.float32)


def _onehot_body(idx_ref, oh_ref):
    R, V = oh_ref.shape
    idxv = idx_ref[:, 0:1]
    cols = lax.broadcasted_iota(jnp.int32, (R, V), 1)
    oh_ref[:, :] = jnp.where(cols == idxv, np.float32(1.0), np.float32(0.0))


def kernel(logits):
    B, S, V = logits.shape
    lg = logits.reshape(S, V)
    R = 8
    idx, ids = pl.pallas_call(
        _argmax_body,
        grid=(S // R,),
        in_specs=[pl.BlockSpec((R, V), lambda i: (i, 0))],
        out_specs=[
            pl.BlockSpec((R, 128), lambda i: (i, 0)),
            pl.BlockSpec((R, 128), lambda i: (i, 0)),
        ],
        out_shape=[
            jax.ShapeDtypeStruct((S, 128), jnp.int32),
            jax.ShapeDtypeStruct((S, 128), jnp.float32),
        ],
        compiler_params=pltpu.CompilerParams(
            dimension_semantics=("parallel",)
        ),
    )(lg)
    R2 = 16
    onehot = pl.pallas_call(
        _onehot_body,
        grid=(S // R2,),
        in_specs=[pl.BlockSpec((R2, 128), lambda i: (i, 0))],
        out_specs=pl.BlockSpec((R2, V), lambda i: (i, 0)),
        out_shape=jax.ShapeDtypeStruct((S, V), jnp.float32),
        compiler_params=pltpu.CompilerParams(
            dimension_semantics=("parallel",)
        ),
    )(idx)
    return ids[:, 0].reshape(B, S), onehot.reshape(B, S, V)


# speedup vs baseline: 1.9589x; 1.0075x over previous
import jax
import jax.numpy as jnp
import numpy as np
from jax import lax
from jax.experimental import pallas as pl
from jax.experimental.pallas import tpu as pltpu

_KS0 = np.uint32(0)
_KS1 = np.uint32(42)
_KS2 = np.uint32(42 ^ 0x1BD11BDA)
_ROT_A = (13, 15, 26, 6)
_ROT_B = (17, 29, 16, 24)
_UNIF_LO = np.float32(1e-9)
_UNIF_SCALE = np.float32(np.float32(1.0) - np.float32(1e-9))


def _rotl(x, d):
    return (x << np.uint32(d)) | (x >> np.uint32(32 - d))


def _gumbel_from_counter(f):
    x0 = jnp.zeros_like(f) + _KS0
    x1 = f + _KS1
    kseq = ((_KS1, _KS2), (_KS2, _KS0), (_KS0, _KS1), (_KS1, _KS2), (_KS2, _KS0))
    rots = (_ROT_A, _ROT_B, _ROT_A, _ROT_B, _ROT_A)
    for i in range(5):
        for r in rots[i]:
            x0 = x0 + x1
            x1 = _rotl(x1, r)
            x1 = x0 ^ x1
        a, b = kseq[i]
        x0 = x0 + a
        x1 = x1 + b + np.uint32(i + 1)
    bits = x0 ^ x1
    fl = lax.bitcast_convert_type(
        (bits >> np.uint32(9)) | np.uint32(0x3F800000), jnp.float32
    ) - np.float32(1.0)
    u = fl + _UNIF_LO
    return -jnp.log(-jnp.log(u))


_CW = 512


def _argmax_body(lg_ref, idx_ref, ids_ref):
    i = pl.program_id(0)
    R, V = lg_ref.shape
    nc = (V + _CW - 1) // _CW
    base = (i * np.int32(R * V)).astype(jnp.uint32)
    fbase = (
        lax.broadcasted_iota(jnp.uint32, (R, _CW), 0) * np.uint32(V)
        + lax.broadcasted_iota(jnp.uint32, (R, _CW), 1)
        + base
    )
    ciota_i = lax.broadcasted_iota(jnp.int32, (R, _CW), 1)

    vmax = jnp.full((R, _CW), -jnp.inf, jnp.float32)
    vidx = jnp.zeros((R, _CW), jnp.int32)
    for c in range(nc):
        cb = min(c * _CW, V - _CW)
        y = lg_ref[:, cb : cb + _CW] + _gumbel_from_counter(fbase + np.uint32(cb))
        upd = y > vmax
        vmax = jnp.where(upd, y, vmax)
        vidx = jnp.where(upd, ciota_i + np.int32(cb), vidx)
    m = jnp.max(vmax, axis=1, keepdims=True)
    idx = jnp.min(
        jnp.where(vmax == m, vidx, np.int32(2**30)), axis=1, keepdims=True
    )
    idx_ref[:, :] = jnp.broadcast_to(idx, idx_ref.shape)
    ids_ref[:, :] = jnp.broadcast_to(idx, ids_ref.shape).astype(jnp.float32)


def _onehot_body(idx_ref, oh_ref):
    R, V = oh_ref.shape
    idxv = idx_ref[:, 0:1]
    cols = lax.broadcasted_iota(jnp.int32, (R, V), 1)
    oh_ref[:, :] = jnp.where(cols == idxv, np.float32(1.0), np.float32(0.0))


def kernel(logits):
    B, S, V = logits.shape
    lg = logits.reshape(S, V)
    R = 8
    idx, ids = pl.pallas_call(
        _argmax_body,
        grid=(S // R,),
        in_specs=[pl.BlockSpec((R, V), lambda i: (i, 0))],
        out_specs=[
            pl.BlockSpec((R, 128), lambda i: (i, 0)),
            pl.BlockSpec((R, 128), lambda i: (i, 0)),
        ],
        out_shape=[
            jax.ShapeDtypeStruct((S, 128), jnp.int32),
            jax.ShapeDtypeStruct((S, 128), jnp.float32),
        ],
        compiler_params=pltpu.CompilerParams(
            dimension_semantics=("parallel",)
        ),
    )(lg)
    R2 = 16
    onehot = pl.pallas_call(
        _onehot_body,
        grid=(S // R2,),
        in_specs=[pl.BlockSpec((R2, 128), lambda i: (i, 0))],
        out_specs=pl.BlockSpec((R2, V), lambda i: (i, 0)),
        out_shape=jax.ShapeDtypeStruct((S, V), jnp.float32),
        compiler_params=pltpu.CompilerParams(
            dimension_semantics=("parallel",)
        ),
    )(idx)
    return ids[:, 0].reshape(B, S), onehot.reshape(B, S, V)
